# Initial kernel scaffold; baseline (speedup 1.0000x reference)
import jax
import jax.numpy as jnp
from jax.experimental import pallas as pl


def kernel(x, edge_index, W1, b1, W2, b2, W3, b3):
    raise NotImplementedError("write your pallas kernel here")


# speedup vs baseline: 5.0774x; 5.0774x over previous
import functools

import jax
import jax.numpy as jnp
from jax import lax
from jax.experimental import pallas as pl
from jax.experimental.pallas import tpu as pltpu
from jax.experimental.pallas import tpu_sc as plsc

NC, NS = 2, 16
BLK = 128
F = 128


def _sc_mesh():
    return plsc.VectorSubcoreMesh(core_axis_name="c", subcore_axis_name="s")


def _make_agg_kernel(P, NBLK, n_pad):
    rows_per_tile = n_pad // NS
    PPC = P // NC
    ZCH = rows_per_tile // 8
    assert ZCH % 8 == 0

    @functools.partial(
        pl.kernel,
        out_type=jax.ShapeDtypeStruct((P, n_pad, F), jnp.float32),
        mesh=_sc_mesh(),
        scratch_types=[
            pltpu.VMEM((NBLK, BLK), jnp.int32),
            pltpu.VMEM((NBLK, BLK), jnp.int32),
            pltpu.VMEM((BLK, F), jnp.float32),
            pltpu.VMEM((ZCH, F), jnp.float32),
            pltpu.VMEM_SHARED((n_pad, F), jnp.float32),
            pltpu.SemaphoreType.DMA,
        ],
    )
    def k(t_hbm, gidx_hbm, sidx_hbm, zeros_hbm, out_hbm, gv, sv, rows_v, zv,
          acc, sem):
        c = lax.axis_index("c")
        s = lax.axis_index("s")
        r0 = s * rows_per_tile

        def zero_acc():
            pltpu.sync_copy(zeros_hbm.at[pl.ds(0, ZCH)], zv)
            for z in range(8):
                pltpu.sync_copy(zv, acc.at[pl.ds(r0 + z * ZCH, ZCH)])

        zero_acc()
        plsc.subcore_barrier()
        for pi in range(PPC):
            p = c * PPC + pi
            pltpu.sync_copy(gidx_hbm.at[p, s], gv)
            pltpu.sync_copy(sidx_hbm.at[p, s], sv)

            def body(j, carry):
                pltpu.async_copy(t_hbm.at[gv.at[j]], rows_v, sem).wait()
                pltpu.sync_copy(rows_v, acc.at[sv.at[j]], add=True)
                return carry

            lax.fori_loop(0, NBLK, body, 0)
            plsc.subcore_barrier()
            for z in range(8):
                pltpu.sync_copy(acc.at[pl.ds(r0 + z * ZCH, ZCH)], zv)
                pltpu.sync_copy(zv, out_hbm.at[p, pl.ds(r0 + z * ZCH, ZCH)])
            if pi + 1 < PPC:
                zero_acc()
                plsc.subcore_barrier()
    return k


def _make_deg_kernel(NBLK, n_pad):
    seg = 4 * n_pad // NS

    @functools.partial(
        pl.kernel,
        out_type=jax.ShapeDtypeStruct((NC * 4 * n_pad,), jnp.float32),
        mesh=_sc_mesh(),
        scratch_types=[
            pltpu.VMEM((NBLK, BLK), jnp.int32),
            pltpu.VMEM((BLK,), jnp.float32),
            pltpu.VMEM((seg,), jnp.float32),
            pltpu.VMEM_SHARED((4 * n_pad,), jnp.float32),
        ],
    )
    def k(degidx_hbm, ones_hbm, zeros_hbm, out_hbm, dv, ones_v, zv, dacc):
        c = lax.axis_index("c")
        s = lax.axis_index("s")
        o0 = s * seg
        pltpu.sync_copy(zeros_hbm.at[pl.ds(o0, seg)], zv)
        pltpu.sync_copy(zv, dacc.at[pl.ds(o0, seg)])
        pltpu.sync_copy(ones_hbm, ones_v)
        plsc.subcore_barrier()
        for tl in range(4):
            t = c * 4 + tl
            pltpu.sync_copy(degidx_hbm.at[t, s], dv)

            def body(j, carry):
                pltpu.sync_copy(ones_v, dacc.at[dv.at[j]], add=True)
                return carry

            lax.fori_loop(0, NBLK, body, 0)
        plsc.subcore_barrier()
        pltpu.sync_copy(dacc.at[pl.ds(o0, seg)], zv)
        pltpu.sync_copy(zv, out_hbm.at[pl.ds(c * (4 * n_pad) + o0, seg)])
    return k


def _rs(deg_row):
    return lax.rsqrt(jnp.maximum(deg_row, 1.0))


def _tc1_body(x_ref, deg_ref, o_ref):
    x = x_ref[...]
    for r in range(4):
        sc = _rs(deg_ref[:, 2 * r])
        o_ref[:, r * 128:(r + 1) * 128] = x * sc[:, None]


def _tc2_body(a_ref, deg_ref, w_ref, b_ref, o_ref):
    bn = a_ref.shape[1]
    acc = jnp.broadcast_to(jnp.sum(b_ref[...], axis=0)[None, :], (bn, 256))
    for r in range(4):
        din = _rs(deg_ref[:, 2 * r + 1])
        a = a_ref[r] * din[:, None]
        acc = acc + jnp.dot(a, w_ref[r], preferred_element_type=jnp.float32)
    h = jnp.maximum(acc, 0.0)
    for r in range(4):
        dsrc = _rs(deg_ref[:, 2 * r])
        o_ref[:, r * 256:(r + 1) * 256] = h * dsrc[:, None]


def _tc3_body(a_ref, deg_ref, w2_ref, b2_ref, w3_ref, o_ref):
    bn = a_ref.shape[1]
    acc = jnp.broadcast_to(jnp.sum(b2_ref[...], axis=0)[None, :], (bn, 256))
    for r in range(4):
        din = _rs(deg_ref[:, 2 * r + 1])
        a = jnp.concatenate([a_ref[2 * r], a_ref[2 * r + 1]], axis=1)
        acc = acc + jnp.dot(a * din[:, None], w2_ref[r],
                            preferred_element_type=jnp.float32)
    h = jnp.maximum(acc, 0.0)
    for r in range(4):
        dsrc = _rs(deg_ref[:, 2 * r])
        o_ref[:, r * 128:(r + 1) * 128] = jnp.dot(
            h * dsrc[:, None], w3_ref[r], preferred_element_type=jnp.float32)


def _tc4_body(a_ref, deg_ref, b3_ref, o_ref):
    bn = a_ref.shape[1]
    acc = jnp.broadcast_to(jnp.sum(b3_ref[...], axis=0)[None, :], (bn, 128))
    for r in range(4):
        din = _rs(deg_ref[:, 2 * r + 1])
        acc = acc + a_ref[r] * din[:, None]
    o_ref[...] = acc


def kernel(x, edge_index, W1, b1, W2, b2, W3, b3):
    N, d_in = x.shape
    R, _, E = edge_index.shape
    assert R == 4 and d_in == 128
    n_pad = (N // F + 2) * F
    e_pad = -(-E // (NS * BLK)) * NS * BLK
    NBLK = e_pad // (NS * BLK)
    pad_n = e_pad - E
    dummy = n_pad - N

    src = edge_index[:, 0, :]
    dst = edge_index[:, 1, :]
    pad_real = (jnp.arange(pad_n, dtype=jnp.int32) * 97) % N
    pad_dummy = N + jnp.arange(pad_n, dtype=jnp.int32) % dummy
    srcg = jnp.concatenate([src, jnp.broadcast_to(pad_real, (R, pad_n))], axis=1)
    dstg = jnp.concatenate([dst, jnp.broadcast_to(pad_dummy, (R, pad_n))], axis=1)
    srcd = jnp.concatenate([src, jnp.broadcast_to(pad_dummy, (R, pad_n))], axis=1)

    rr = jnp.arange(R, dtype=jnp.int32)
    g4 = (srcg * 4 + rr[:, None]).reshape(R, NS, NBLK, BLK)
    s4 = dstg.reshape(R, NS, NBLK, BLK)
    ch = jnp.arange(2, dtype=jnp.int32)
    g8 = (srcg[:, None, :] * 8 + (rr[:, None, None] * 2 + ch[None, :, None])
          ).reshape(2 * R, NS, NBLK, BLK)
    s8 = jnp.broadcast_to(dstg[:, None, :], (R, 2, e_pad)).reshape(
        2 * R, NS, NBLK, BLK)
    dtasks = []
    for r in range(R):
        for base in (srcd[r], dstg[r]):
            t = len(dtasks)
            dtasks.append(base + (t % 4) * n_pad)
    degidx = jnp.stack(dtasks).reshape(2 * R, NS, NBLK, BLK)

    zeros2d = jnp.zeros((n_pad, F), jnp.float32)
    zeros1d = jnp.zeros((4 * n_pad,), jnp.float32)
    ones128 = jnp.ones((BLK,), jnp.float32)

    deg = _make_deg_kernel(NBLK, n_pad)(degidx, ones128, zeros1d)
    deg = deg.reshape(2 * R, n_pad).T

    x_pad = jnp.concatenate([x, jnp.zeros((n_pad - N, d_in), x.dtype)], axis=0)

    bn = n_pad // 16
    grid = 16
    deg_spec = pl.BlockSpec((bn, 8), lambda i: (i, 0))

    def full_spec(shp):
        return pl.BlockSpec(shp, lambda i, _n=len(shp): (0,) * _n)

    t1 = pl.pallas_call(
        _tc1_body,
        grid=(grid,),
        in_specs=[pl.BlockSpec((bn, 128), lambda i: (i, 0)), deg_spec],
        out_specs=pl.BlockSpec((bn, 512), lambda i: (i, 0)),
        out_shape=jax.ShapeDtypeStruct((n_pad, 512), jnp.float32),
    )(x_pad, deg)

    agg4 = _make_agg_kernel(4, NBLK, n_pad)
    agg8 = _make_agg_kernel(8, NBLK, n_pad)

    a1 = agg4(t1.reshape(n_pad * 4, F), g4, s4, zeros2d)

    t2 = pl.pallas_call(
        _tc2_body,
        grid=(grid,),
        in_specs=[pl.BlockSpec((4, bn, 128), lambda i: (0, i, 0)), deg_spec,
                  full_spec((4, 128, 256)), full_spec((4, 256))],
        out_specs=pl.BlockSpec((bn, 1024), lambda i: (i, 0)),
        out_shape=jax.ShapeDtypeStruct((n_pad, 1024), jnp.float32),
    )(a1, deg, W1, b1)

    a2 = agg8(t2.reshape(n_pad * 8, F), g8, s8, zeros2d)

    t3 = pl.pallas_call(
        _tc3_body,
        grid=(grid,),
        in_specs=[pl.BlockSpec((8, bn, 128), lambda i: (0, i, 0)), deg_spec,
                  full_spec((4, 256, 256)), full_spec((4, 256)),
                  full_spec((4, 256, 128))],
        out_specs=pl.BlockSpec((bn, 512), lambda i: (i, 0)),
        out_shape=jax.ShapeDtypeStruct((n_pad, 512), jnp.float32),
    )(a2, deg, W2, b2, W3)

    a3 = agg4(t3.reshape(n_pad * 4, F), g4, s4, zeros2d)

    y = pl.pallas_call(
        _tc4_body,
        grid=(grid,),
        in_specs=[pl.BlockSpec((4, bn, 128), lambda i: (0, i, 0)), deg_spec,
                  full_spec((4, 128))],
        out_specs=pl.BlockSpec((bn, 128), lambda i: (i, 0)),
        out_shape=jax.ShapeDtypeStruct((n_pad, 128), jnp.float32),
    )(a3, deg, b3)

    return y[:N]


# speedup vs baseline: 5.7167x; 1.1259x over previous
import functools

import jax
import jax.numpy as jnp
from jax import lax
from jax.experimental import pallas as pl
from jax.experimental.pallas import tpu as pltpu
from jax.experimental.pallas import tpu_sc as plsc

NC, NS = 2, 16
BLK = 128
F = 128


def _sc_mesh():
    return plsc.VectorSubcoreMesh(core_axis_name="c", subcore_axis_name="s")


def _make_agg_kernel(P, NBLK, n_pad):
    rows_per_tile = n_pad // NS
    PPC = P // NC
    NZ = 16
    ZCH = rows_per_tile // NZ
    assert ZCH % 8 == 0 and NBLK % 2 == 0

    @functools.partial(
        pl.kernel,
        out_type=jax.ShapeDtypeStruct((P, n_pad, F), jnp.float32),
        mesh=_sc_mesh(),
        scratch_types=[
            pltpu.VMEM((NBLK, BLK), jnp.int32),
            pltpu.VMEM((NBLK, BLK), jnp.int32),
            pltpu.VMEM((2, BLK, F), jnp.float32),
            pltpu.VMEM((ZCH, F), jnp.float32),
            pltpu.VMEM_SHARED((n_pad, F), jnp.float32),
            pltpu.SemaphoreType.DMA,
            pltpu.SemaphoreType.DMA,
            pltpu.SemaphoreType.DMA,
            pltpu.SemaphoreType.DMA,
        ],
    )
    def k(t_hbm, gidx_hbm, sidx_hbm, zeros_hbm, out_hbm, gv, sv, rows_v, zv,
          acc, sg0, sg1, ss0, ss1):
        c = lax.axis_index("c")
        s = lax.axis_index("s")
        r0 = s * rows_per_tile

        def zero_acc():
            pltpu.sync_copy(zeros_hbm.at[pl.ds(0, ZCH)], zv)
            for z in range(NZ):
                pltpu.sync_copy(zv, acc.at[pl.ds(r0 + z * ZCH, ZCH)])

        zero_acc()
        plsc.subcore_barrier()
        for pi in range(PPC):
            p = c * PPC + pi
            pltpu.sync_copy(gidx_hbm.at[p, s], gv)
            pltpu.sync_copy(sidx_hbm.at[p, s], sv)

            def gath(j, b, sem):
                pltpu.async_copy(t_hbm.at[gv.at[j]], rows_v.at[b], sem)

            def wait_gath(j, b, sem):
                pltpu.make_async_copy(t_hbm.at[gv.at[j]], rows_v.at[b],
                                      sem).wait()

            def scat(j, b, sem):
                pltpu.async_copy(rows_v.at[b], acc.at[sv.at[j]], sem, add=True)

            def wait_scat(j, b, sem):
                pltpu.make_async_copy(rows_v.at[b], acc.at[sv.at[j]],
                                      sem).wait()

            gath(0, 0, sg0)
            gath(1, 1, sg1)

            def pair(m, carry):
                j0 = 2 * m
                wait_gath(j0, 0, sg0)
                scat(j0, 0, ss0)
                wait_gath(j0 + 1, 1, sg1)
                scat(j0 + 1, 1, ss1)
                wait_scat(j0, 0, ss0)
                gath(j0 + 2, 0, sg0)
                wait_scat(j0 + 1, 1, ss1)
                gath(j0 + 3, 1, sg1)
                return carry

            lax.fori_loop(0, NBLK // 2 - 1, pair, 0)
            jl = NBLK - 2
            wait_gath(jl, 0, sg0)
            pltpu.sync_copy(rows_v.at[0], acc.at[sv.at[jl]], add=True)
            wait_gath(jl + 1, 1, sg1)
            pltpu.sync_copy(rows_v.at[1], acc.at[sv.at[jl + 1]], add=True)
            plsc.subcore_barrier()
            for z in range(NZ):
                pltpu.sync_copy(acc.at[pl.ds(r0 + z * ZCH, ZCH)], zv)
                pltpu.sync_copy(zv, out_hbm.at[p, pl.ds(r0 + z * ZCH, ZCH)])
            if pi + 1 < PPC:
                zero_acc()
                plsc.subcore_barrier()
    return k


def _make_deg_kernel(NBLK, n_pad):
    seg = 4 * n_pad // NS

    @functools.partial(
        pl.kernel,
        out_type=jax.ShapeDtypeStruct((NC * 4 * n_pad,), jnp.float32),
        mesh=_sc_mesh(),
        scratch_types=[
            pltpu.VMEM((NBLK, BLK), jnp.int32),
            pltpu.VMEM((BLK,), jnp.float32),
            pltpu.VMEM((seg,), jnp.float32),
            pltpu.VMEM_SHARED((4 * n_pad,), jnp.float32),
        ],
    )
    def k(degidx_hbm, ones_hbm, zeros_hbm, out_hbm, dv, ones_v, zv, dacc):
        c = lax.axis_index("c")
        s = lax.axis_index("s")
        o0 = s * seg
        pltpu.sync_copy(zeros_hbm.at[pl.ds(o0, seg)], zv)
        pltpu.sync_copy(zv, dacc.at[pl.ds(o0, seg)])
        pltpu.sync_copy(ones_hbm, ones_v)
        plsc.subcore_barrier()
        for tl in range(4):
            t = c * 4 + tl
            pltpu.sync_copy(degidx_hbm.at[t, s], dv)

            def body(j, carry):
                pltpu.sync_copy(ones_v, dacc.at[dv.at[j]], add=True)
                return carry

            lax.fori_loop(0, NBLK, body, 0)
        plsc.subcore_barrier()
        pltpu.sync_copy(dacc.at[pl.ds(o0, seg)], zv)
        pltpu.sync_copy(zv, out_hbm.at[pl.ds(c * (4 * n_pad) + o0, seg)])
    return k


def _rs(deg_row):
    return lax.rsqrt(jnp.maximum(deg_row, 1.0))


def _tc1_body(x_ref, deg_ref, o_ref):
    x = x_ref[...]
    for r in range(4):
        sc = _rs(deg_ref[:, 2 * r])
        o_ref[:, r * 128:(r + 1) * 128] = x * sc[:, None]


def _tc2_body(a_ref, deg_ref, w_ref, b_ref, o_ref):
    bn = a_ref.shape[1]
    acc = jnp.broadcast_to(jnp.sum(b_ref[...], axis=0)[None, :], (bn, 256))
    for r in range(4):
        din = _rs(deg_ref[:, 2 * r + 1])
        a = a_ref[r] * din[:, None]
        acc = acc + jnp.dot(a, w_ref[r], preferred_element_type=jnp.float32)
    h = jnp.maximum(acc, 0.0)
    for r in range(4):
        dsrc = _rs(deg_ref[:, 2 * r])
        o_ref[:, r * 256:(r + 1) * 256] = h * dsrc[:, None]


def _tc3_body(a_ref, deg_ref, w2_ref, b2_ref, w3_ref, o_ref):
    bn = a_ref.shape[1]
    acc = jnp.broadcast_to(jnp.sum(b2_ref[...], axis=0)[None, :], (bn, 256))
    for r in range(4):
        din = _rs(deg_ref[:, 2 * r + 1])
        a = jnp.concatenate([a_ref[2 * r], a_ref[2 * r + 1]], axis=1)
        acc = acc + jnp.dot(a * din[:, None], w2_ref[r],
                            preferred_element_type=jnp.float32)
    h = jnp.maximum(acc, 0.0)
    for r in range(4):
        dsrc = _rs(deg_ref[:, 2 * r])
        o_ref[:, r * 128:(r + 1) * 128] = jnp.dot(
            h * dsrc[:, None], w3_ref[r], preferred_element_type=jnp.float32)


def _tc4_body(a_ref, deg_ref, b3_ref, o_ref):
    bn = a_ref.shape[1]
    acc = jnp.broadcast_to(jnp.sum(b3_ref[...], axis=0)[None, :], (bn, 128))
    for r in range(4):
        din = _rs(deg_ref[:, 2 * r + 1])
        acc = acc + a_ref[r] * din[:, None]
    o_ref[...] = acc


def kernel(x, edge_index, W1, b1, W2, b2, W3, b3):
    N, d_in = x.shape
    R, _, E = edge_index.shape
    assert R == 4 and d_in == 128
    n_pad = (N // F + 2) * F
    e_pad = -(-E // (NS * BLK)) * NS * BLK
    NBLK = e_pad // (NS * BLK)
    pad_n = e_pad - E
    dummy = n_pad - N

    src = edge_index[:, 0, :]
    dst = edge_index[:, 1, :]
    pad_real = (jnp.arange(pad_n, dtype=jnp.int32) * 97) % N
    pad_dummy = N + jnp.arange(pad_n, dtype=jnp.int32) % dummy
    srcg = jnp.concatenate([src, jnp.broadcast_to(pad_real, (R, pad_n))], axis=1)
    dstg = jnp.concatenate([dst, jnp.broadcast_to(pad_dummy, (R, pad_n))], axis=1)
    srcd = jnp.concatenate([src, jnp.broadcast_to(pad_dummy, (R, pad_n))], axis=1)

    rr = jnp.arange(R, dtype=jnp.int32)
    g4 = (srcg * 4 + rr[:, None]).reshape(R, NS, NBLK, BLK)
    s4 = dstg.reshape(R, NS, NBLK, BLK)
    ch = jnp.arange(2, dtype=jnp.int32)
    g8 = (srcg[:, None, :] * 8 + (rr[:, None, None] * 2 + ch[None, :, None])
          ).reshape(2 * R, NS, NBLK, BLK)
    s8 = jnp.broadcast_to(dstg[:, None, :], (R, 2, e_pad)).reshape(
        2 * R, NS, NBLK, BLK)
    dtasks = []
    for r in range(R):
        for base in (srcd[r], dstg[r]):
            t = len(dtasks)
            dtasks.append(base + (t % 4) * n_pad)
    degidx = jnp.stack(dtasks).reshape(2 * R, NS, NBLK, BLK)

    zeros2d = jnp.zeros((n_pad, F), jnp.float32)
    zeros1d = jnp.zeros((4 * n_pad,), jnp.float32)
    ones128 = jnp.ones((BLK,), jnp.float32)

    deg = _make_deg_kernel(NBLK, n_pad)(degidx, ones128, zeros1d)
    deg = deg.reshape(2 * R, n_pad).T

    x_pad = jnp.concatenate([x, jnp.zeros((n_pad - N, d_in), x.dtype)], axis=0)

    bn = n_pad // 16
    grid = 16
    deg_spec = pl.BlockSpec((bn, 8), lambda i: (i, 0))

    def full_spec(shp):
        return pl.BlockSpec(shp, lambda i, _n=len(shp): (0,) * _n)

    t1 = pl.pallas_call(
        _tc1_body,
        grid=(grid,),
        in_specs=[pl.BlockSpec((bn, 128), lambda i: (i, 0)), deg_spec],
        out_specs=pl.BlockSpec((bn, 512), lambda i: (i, 0)),
        out_shape=jax.ShapeDtypeStruct((n_pad, 512), jnp.float32),
    )(x_pad, deg)

    agg4 = _make_agg_kernel(4, NBLK, n_pad)
    agg8 = _make_agg_kernel(8, NBLK, n_pad)

    a1 = agg4(t1.reshape(n_pad * 4, F), g4, s4, zeros2d)

    t2 = pl.pallas_call(
        _tc2_body,
        grid=(grid,),
        in_specs=[pl.BlockSpec((4, bn, 128), lambda i: (0, i, 0)), deg_spec,
                  full_spec((4, 128, 256)), full_spec((4, 256))],
        out_specs=pl.BlockSpec((bn, 1024), lambda i: (i, 0)),
        out_shape=jax.ShapeDtypeStruct((n_pad, 1024), jnp.float32),
    )(a1, deg, W1, b1)

    a2 = agg8(t2.reshape(n_pad * 8, F), g8, s8, zeros2d)

    t3 = pl.pallas_call(
        _tc3_body,
        grid=(grid,),
        in_specs=[pl.BlockSpec((8, bn, 128), lambda i: (0, i, 0)), deg_spec,
                  full_spec((4, 256, 256)), full_spec((4, 256)),
                  full_spec((4, 256, 128))],
        out_specs=pl.BlockSpec((bn, 512), lambda i: (i, 0)),
        out_shape=jax.ShapeDtypeStruct((n_pad, 512), jnp.float32),
    )(a2, deg, W2, b2, W3)

    a3 = agg4(t3.reshape(n_pad * 4, F), g4, s4, zeros2d)

    y = pl.pallas_call(
        _tc4_body,
        grid=(grid,),
        in_specs=[pl.BlockSpec((4, bn, 128), lambda i: (0, i, 0)), deg_spec,
                  full_spec((4, 128))],
        out_specs=pl.BlockSpec((bn, 128), lambda i: (i, 0)),
        out_shape=jax.ShapeDtypeStruct((n_pad, 128), jnp.float32),
    )(a3, deg, b3)

    return y[:N]


# speedup vs baseline: 6.0923x; 1.0657x over previous
import functools

import jax
import jax.numpy as jnp
from jax import lax
from jax.experimental import pallas as pl
from jax.experimental.pallas import tpu as pltpu
from jax.experimental.pallas import tpu_sc as plsc

NC, NS = 2, 16
BLK = 128
F = 128


def _sc_mesh():
    return plsc.VectorSubcoreMesh(core_axis_name="c", subcore_axis_name="s")


def _make_agg_kernel(P, NBLK, n_pad):
    rows_per_tile = n_pad // NS
    PPC = P // NC
    NZ = 16
    ZCH = rows_per_tile // NZ
    assert ZCH % 8 == 0 and NBLK % 2 == 0

    @functools.partial(
        pl.kernel,
        out_type=jax.ShapeDtypeStruct((P, n_pad, F), jnp.float32),
        mesh=_sc_mesh(),
        scratch_types=[
            pltpu.VMEM((NBLK, BLK), jnp.int32),
            pltpu.VMEM((NBLK, BLK), jnp.int32),
            pltpu.VMEM((2, BLK, F), jnp.float32),
            pltpu.VMEM((ZCH, F), jnp.float32),
            pltpu.VMEM_SHARED((n_pad, F), jnp.float32),
            pltpu.SemaphoreType.DMA,
            pltpu.SemaphoreType.DMA,
            pltpu.SemaphoreType.DMA,
            pltpu.SemaphoreType.DMA,
        ],
    )
    def k(t_hbm, gidx_hbm, sidx_hbm, zeros_hbm, out_hbm, gv, sv, rows_v, zv,
          acc, sg0, sg1, ss0, ss1):
        c = lax.axis_index("c")
        s = lax.axis_index("s")
        r0 = s * rows_per_tile

        def gath(j, b, sem):
            pltpu.async_copy(t_hbm.at[gv.at[j]], rows_v.at[b], sem)

        def wait_gath(j, b, sem):
            pltpu.make_async_copy(t_hbm.at[gv.at[j]], rows_v.at[b],
                                  sem).wait()

        def scat(j, b, sem):
            pltpu.async_copy(rows_v.at[b], acc.at[sv.at[j]], sem, add=True)

        def wait_scat(j, b, sem):
            pltpu.make_async_copy(rows_v.at[b], acc.at[sv.at[j]],
                                  sem).wait()

        def zero_acc():
            for z in range(NZ):
                pltpu.sync_copy(zv, acc.at[pl.ds(r0 + z * ZCH, ZCH)])

        def load_and_prime(pi_next):
            p = c * PPC + pi_next
            pltpu.sync_copy(gidx_hbm.at[p, s], gv)
            pltpu.sync_copy(sidx_hbm.at[p, s], sv)
            gath(0, 0, sg0)
            gath(1, 1, sg1)

        NDCH = rows_per_tile // BLK

        def dump_out(p):
            for z in range(NDCH):
                b = z & 1
                if z >= 2:
                    pltpu.make_async_copy(
                        rows_v.at[b],
                        out_hbm.at[p, pl.ds(r0 + (z - 2) * BLK, BLK)],
                        ss0 if b == 0 else ss1).wait()
                pltpu.sync_copy(acc.at[pl.ds(r0 + z * BLK, BLK)],
                                rows_v.at[b])
                pltpu.async_copy(rows_v.at[b],
                                 out_hbm.at[p, pl.ds(r0 + z * BLK, BLK)],
                                 ss0 if b == 0 else ss1)
            for z in (NDCH - 2, NDCH - 1):
                b = z & 1
                pltpu.make_async_copy(
                    rows_v.at[b],
                    out_hbm.at[p, pl.ds(r0 + z * BLK, BLK)],
                    ss0 if b == 0 else ss1).wait()

        pltpu.sync_copy(zeros_hbm.at[pl.ds(0, ZCH)], zv)
        load_and_prime(0)
        zero_acc()
        plsc.subcore_barrier()
        for pi in range(PPC):
            p = c * PPC + pi

            def pair(m, carry):
                j0 = 2 * m
                wait_gath(j0, 0, sg0)
                scat(j0, 0, ss0)
                wait_gath(j0 + 1, 1, sg1)
                scat(j0 + 1, 1, ss1)
                wait_scat(j0, 0, ss0)
                gath(j0 + 2, 0, sg0)
                wait_scat(j0 + 1, 1, ss1)
                gath(j0 + 3, 1, sg1)
                return carry

            lax.fori_loop(0, NBLK // 2 - 1, pair, 0)
            jl = NBLK - 2
            wait_gath(jl, 0, sg0)
            pltpu.sync_copy(rows_v.at[0], acc.at[sv.at[jl]], add=True)
            wait_gath(jl + 1, 1, sg1)
            pltpu.sync_copy(rows_v.at[1], acc.at[sv.at[jl + 1]], add=True)
            plsc.subcore_barrier()
            dump_out(p)
            if pi + 1 < PPC:
                load_and_prime(pi + 1)
                zero_acc()
                plsc.subcore_barrier()
    return k


def _make_deg_kernel(NBLK, n_pad):
    seg = 4 * n_pad // NS

    @functools.partial(
        pl.kernel,
        out_type=jax.ShapeDtypeStruct((NC * 4 * n_pad,), jnp.float32),
        mesh=_sc_mesh(),
        scratch_types=[
            pltpu.VMEM((NBLK, BLK), jnp.int32),
            pltpu.VMEM((BLK,), jnp.float32),
            pltpu.VMEM((seg,), jnp.float32),
            pltpu.VMEM_SHARED((4 * n_pad,), jnp.float32),
        ],
    )
    def k(degidx_hbm, ones_hbm, zeros_hbm, out_hbm, dv, ones_v, zv, dacc):
        c = lax.axis_index("c")
        s = lax.axis_index("s")
        o0 = s * seg
        pltpu.sync_copy(zeros_hbm.at[pl.ds(o0, seg)], zv)
        pltpu.sync_copy(zv, dacc.at[pl.ds(o0, seg)])
        pltpu.sync_copy(ones_hbm, ones_v)
        plsc.subcore_barrier()
        for tl in range(4):
            t = c * 4 + tl
            pltpu.sync_copy(degidx_hbm.at[t, s], dv)

            def body(j, carry):
                pltpu.sync_copy(ones_v, dacc.at[dv.at[j]], add=True)
                return carry

            lax.fori_loop(0, NBLK, body, 0)
        plsc.subcore_barrier()
        pltpu.sync_copy(dacc.at[pl.ds(o0, seg)], zv)
        pltpu.sync_copy(zv, out_hbm.at[pl.ds(c * (4 * n_pad) + o0, seg)])
    return k


def _rs(deg_row):
    return lax.rsqrt(jnp.maximum(deg_row, 1.0))


def _tc1_body(x_ref, deg_ref, o_ref):
    x = x_ref[...]
    for r in range(4):
        sc = _rs(deg_ref[:, 2 * r])
        o_ref[:, r * 128:(r + 1) * 128] = x * sc[:, None]


def _tc2_body(a_ref, deg_ref, w_ref, b_ref, o_ref):
    bn = a_ref.shape[1]
    acc = jnp.broadcast_to(jnp.sum(b_ref[...], axis=0)[None, :], (bn, 256))
    for r in range(4):
        din = _rs(deg_ref[:, 2 * r + 1])
        acc = acc + din[:, None] * jnp.dot(
            a_ref[r].astype(jnp.bfloat16), w_ref[r],
            preferred_element_type=jnp.float32)
    h = jnp.maximum(acc, 0.0)
    for r in range(4):
        dsrc = _rs(deg_ref[:, 2 * r])
        o_ref[:, r * 256:(r + 1) * 256] = h * dsrc[:, None]


def _tc3_body(a_ref, deg_ref, w2_ref, b2_ref, w3_ref, o_ref):
    bn = a_ref.shape[1]
    acc = jnp.broadcast_to(jnp.sum(b2_ref[...], axis=0)[None, :], (bn, 256))
    for r in range(4):
        din = _rs(deg_ref[:, 2 * r + 1])
        a = jnp.concatenate([a_ref[2 * r], a_ref[2 * r + 1]],
                            axis=1).astype(jnp.bfloat16)
        acc = acc + din[:, None] * jnp.dot(a, w2_ref[r],
                                           preferred_element_type=jnp.float32)
    h = jnp.maximum(acc, 0.0).astype(jnp.bfloat16)
    for r in range(4):
        dsrc = _rs(deg_ref[:, 2 * r])
        o_ref[:, r * 128:(r + 1) * 128] = dsrc[:, None] * jnp.dot(
            h, w3_ref[r], preferred_element_type=jnp.float32)


def _tc4_body(a_ref, deg_ref, b3_ref, o_ref):
    bn = a_ref.shape[1]
    acc = jnp.broadcast_to(jnp.sum(b3_ref[...], axis=0)[None, :], (bn, 128))
    for r in range(4):
        din = _rs(deg_ref[:, 2 * r + 1])
        acc = acc + a_ref[r].astype(jnp.float32) * din[:, None]
    o_ref[...] = acc


def kernel(x, edge_index, W1, b1, W2, b2, W3, b3):
    N, d_in = x.shape
    R, _, E = edge_index.shape
    assert R == 4 and d_in == 128
    n_pad = (N // F + 2) * F
    e_pad = -(-E // (NS * BLK)) * NS * BLK
    NBLK = e_pad // (NS * BLK)
    pad_n = e_pad - E
    dummy = n_pad - N

    src = edge_index[:, 0, :]
    dst = edge_index[:, 1, :]
    pad_real = (jnp.arange(pad_n, dtype=jnp.int32) * 97) % N
    pad_dummy = N + jnp.arange(pad_n, dtype=jnp.int32) % dummy
    srcg = jnp.concatenate([src, jnp.broadcast_to(pad_real, (R, pad_n))], axis=1)
    dstg = jnp.concatenate([dst, jnp.broadcast_to(pad_dummy, (R, pad_n))], axis=1)
    srcd = jnp.concatenate([src, jnp.broadcast_to(pad_dummy, (R, pad_n))], axis=1)

    rr = jnp.arange(R, dtype=jnp.int32)
    g4 = (srcg * 4 + rr[:, None]).reshape(R, NS, NBLK, BLK)
    s4 = dstg.reshape(R, NS, NBLK, BLK)
    ch = jnp.arange(2, dtype=jnp.int32)
    g8 = (srcg[:, None, :] * 8 + (rr[:, None, None] * 2 + ch[None, :, None])
          ).reshape(2 * R, NS, NBLK, BLK)
    s8 = jnp.broadcast_to(dstg[:, None, :], (R, 2, e_pad)).reshape(
        2 * R, NS, NBLK, BLK)
    dtasks = []
    for r in range(R):
        for base in (srcd[r], dstg[r]):
            t = len(dtasks)
            dtasks.append(base + (t % 4) * n_pad)
    degidx = jnp.stack(dtasks).reshape(2 * R, NS, NBLK, BLK)

    zeros2d = jnp.zeros((n_pad, F), jnp.float32)
    zeros1d = jnp.zeros((4 * n_pad,), jnp.float32)
    ones128 = jnp.ones((BLK,), jnp.float32)

    deg = _make_deg_kernel(NBLK, n_pad)(degidx, ones128, zeros1d)
    deg = deg.reshape(2 * R, n_pad).T

    x_pad = jnp.concatenate([x, jnp.zeros((n_pad - N, d_in), x.dtype)], axis=0)
    W1b = W1.astype(jnp.bfloat16)
    W2b = W2.astype(jnp.bfloat16)
    W3b = W3.astype(jnp.bfloat16)

    bn = n_pad // 16
    grid = 16
    deg_spec = pl.BlockSpec((bn, 8), lambda i: (i, 0))

    def full_spec(shp):
        return pl.BlockSpec(shp, lambda i, _n=len(shp): (0,) * _n)

    t1 = pl.pallas_call(
        _tc1_body,
        grid=(grid,),
        in_specs=[pl.BlockSpec((bn, 128), lambda i: (i, 0)), deg_spec],
        out_specs=pl.BlockSpec((bn, 512), lambda i: (i, 0)),
        out_shape=jax.ShapeDtypeStruct((n_pad, 512), jnp.float32),
    )(x_pad, deg)

    agg4 = _make_agg_kernel(4, NBLK, n_pad)
    agg8 = _make_agg_kernel(8, NBLK, n_pad)

    a1 = agg4(t1.reshape(n_pad * 4, F), g4, s4, zeros2d)

    t2 = pl.pallas_call(
        _tc2_body,
        grid=(grid,),
        in_specs=[pl.BlockSpec((4, bn, 128), lambda i: (0, i, 0)), deg_spec,
                  full_spec((4, 128, 256)), full_spec((4, 256))],
        out_specs=pl.BlockSpec((bn, 1024), lambda i: (i, 0)),
        out_shape=jax.ShapeDtypeStruct((n_pad, 1024), jnp.float32),
    )(a1, deg, W1b, b1)

    a2 = agg8(t2.reshape(n_pad * 8, F), g8, s8, zeros2d)

    t3 = pl.pallas_call(
        _tc3_body,
        grid=(grid,),
        in_specs=[pl.BlockSpec((8, bn, 128), lambda i: (0, i, 0)), deg_spec,
                  full_spec((4, 256, 256)), full_spec((4, 256)),
                  full_spec((4, 256, 128))],
        out_specs=pl.BlockSpec((bn, 512), lambda i: (i, 0)),
        out_shape=jax.ShapeDtypeStruct((n_pad, 512), jnp.float32),
    )(a2, deg, W2b, b2, W3b)

    a3 = agg4(t3.reshape(n_pad * 4, F), g4, s4, zeros2d)

    y = pl.pallas_call(
        _tc4_body,
        grid=(grid,),
        in_specs=[pl.BlockSpec((4, bn, 128), lambda i: (0, i, 0)), deg_spec,
                  full_spec((4, 128))],
        out_specs=pl.BlockSpec((bn, 128), lambda i: (i, 0)),
        out_shape=jax.ShapeDtypeStruct((n_pad, 128), jnp.float32),
    )(a3, deg, b3)

    return y[:N]
